# Initial kernel scaffold; baseline (speedup 1.0000x reference)
import jax
import jax.numpy as jnp
from jax.experimental import pallas as pl


def kernel(input_tokens, table):
    raise NotImplementedError("write your pallas kernel here")


# speedup vs baseline: 1.0946x; 1.0946x over previous
import functools

import jax
import jax.numpy as jnp
from jax import lax
from jax.experimental import pallas as pl
from jax.experimental.pallas import tpu as pltpu
from jax.experimental.pallas import tpu_sc as plsc

_CHUNK = 1024


@functools.partial(jax.jit, static_argnames=("nb", "nc", "ns", "bpw", "nsteps"))
def _sc_gather(flat_idx, table, *, nb, nc, ns, bpw, nsteps):
    D = table.shape[1]
    mesh = plsc.VectorSubcoreMesh(core_axis_name="c", subcore_axis_name="s")

    @functools.partial(
        pl.kernel,
        mesh=mesh,
        out_type=jax.ShapeDtypeStruct((nb, D), jnp.float32),
        scratch_types=[
            pltpu.VMEM((_CHUNK,), jnp.int32),
            pltpu.VMEM((_CHUNK, D), jnp.float32),
            pltpu.SemaphoreType.DMA,
        ],
        compiler_params=pltpu.CompilerParams(use_tc_tiling_on_sc=False),
    )
    def k(idx_hbm, table_hbm, out_hbm, idx_v, rows_v, sem):
        wid = lax.axis_index("s") * nc + lax.axis_index("c")
        base = wid * bpw

        def body(i, carry):
            off = base + i * _CHUNK
            pltpu.sync_copy(idx_hbm.at[pl.ds(off, _CHUNK)], idx_v)
            pltpu.async_copy(table_hbm.at[idx_v], rows_v, sem).wait()
            pltpu.sync_copy(rows_v, out_hbm.at[pl.ds(off, _CHUNK)])
            return carry

        lax.fori_loop(0, nsteps, body, 0, unroll=False)

    return k(flat_idx, table)


def kernel(input_tokens, table):
    B, H = input_tokens.shape
    D = table.shape[1]
    info = plsc.get_sparse_core_info()
    nc, ns = info.num_cores, info.num_subcores
    nb = B * H
    nw = nc * ns
    bpw = nb // nw
    nsteps = bpw // _CHUNK
    flat = input_tokens.reshape(nb).astype(jnp.int32)
    out = _sc_gather(flat, table, nb=nb, nc=nc, ns=ns, bpw=bpw, nsteps=nsteps)
    return out.reshape(B, H, D)


# speedup vs baseline: 1.1126x; 1.0164x over previous
import functools

import jax
import jax.numpy as jnp
from jax import lax
from jax.experimental import pallas as pl
from jax.experimental.pallas import tpu as pltpu
from jax.experimental.pallas import tpu_sc as plsc

_CHUNK = 1280


@functools.partial(jax.jit, static_argnames=("nb", "nc", "ns", "bpw", "npairs"))
def _sc_gather(flat_idx, table, *, nb, nc, ns, bpw, npairs):
    D = table.shape[1]
    C = _CHUNK
    mesh = plsc.VectorSubcoreMesh(core_axis_name="c", subcore_axis_name="s")

    @functools.partial(
        pl.kernel,
        mesh=mesh,
        out_type=jax.ShapeDtypeStruct((nb, D), jnp.float32),
        scratch_types=[
            pltpu.VMEM((bpw,), jnp.int32),
            pltpu.VMEM((C, D), jnp.float32),
            pltpu.VMEM((C, D), jnp.float32),
            pltpu.SemaphoreType.DMA,
            pltpu.SemaphoreType.DMA,
            pltpu.SemaphoreType.DMA,
            pltpu.SemaphoreType.DMA,
        ],
        compiler_params=pltpu.CompilerParams(use_tc_tiling_on_sc=False),
    )
    def k(idx_hbm, table_hbm, out_hbm, idx_all, rows0, rows1, sg0, sg1, sw0, sw1):
        wid = lax.axis_index("s") * nc + lax.axis_index("c")
        base = wid * bpw
        pltpu.sync_copy(idx_hbm.at[pl.ds(base, bpw)], idx_all)

        def gather(j, buf, sem):
            pltpu.async_copy(table_hbm.at[idx_all.at[pl.ds(j * C, C)]], buf, sem)

        def write(j, buf, sem):
            pltpu.async_copy(buf, out_hbm.at[pl.ds(base + j * C, C)], sem)

        def wait_write(j, buf, sem):
            pltpu.make_async_copy(buf, out_hbm.at[pl.ds(base + j * C, C)], sem).wait()

        def wait_gather(buf, sem):
            pltpu.make_async_copy(table_hbm.at[idx_all.at[pl.ds(0, C)]], buf, sem).wait()

        gather(0, rows0, sg0)

        def body(t, carry):
            j0 = 2 * t
            j1 = j0 + 1

            @pl.when(t > 0)
            def _():
                wait_write(j1 - 2, rows1, sw1)

            gather(j1, rows1, sg1)
            wait_gather(rows0, sg0)
            write(j0, rows0, sw0)
            wait_gather(rows1, sg1)
            wait_write(j0, rows0, sw0)

            @pl.when(t < npairs - 1)
            def _():
                gather(j0 + 2, rows0, sg0)

            write(j1, rows1, sw1)
            return carry

        lax.fori_loop(0, npairs, body, 0, unroll=False)
        wait_write(2 * npairs - 1, rows1, sw1)

    return k(flat_idx, table)


def kernel(input_tokens, table):
    B, H = input_tokens.shape
    D = table.shape[1]
    info = plsc.get_sparse_core_info()
    nc, ns = info.num_cores, info.num_subcores
    nb = B * H
    nw = nc * ns
    bpw = nb // nw
    npairs = bpw // (2 * _CHUNK)
    flat = input_tokens.reshape(nb).astype(jnp.int32)
    out = _sc_gather(flat, table, nb=nb, nc=nc, ns=ns, bpw=bpw, npairs=npairs)
    return out.reshape(B, H, D)


# speedup vs baseline: 1.1139x; 1.0012x over previous
import functools

import jax
import jax.numpy as jnp
from jax import lax
from jax.experimental import pallas as pl
from jax.experimental.pallas import tpu as pltpu
from jax.experimental.pallas import tpu_sc as plsc

_CHUNK = 640
_NSLOTS = 4


@functools.partial(jax.jit, static_argnames=("nb", "nc", "ns", "bpw", "nrounds"))
def _sc_gather(flat_idx, table, *, nb, nc, ns, bpw, nrounds):
    D = table.shape[1]
    C = _CHUNK
    N = _NSLOTS
    mesh = plsc.VectorSubcoreMesh(core_axis_name="c", subcore_axis_name="s")

    @functools.partial(
        pl.kernel,
        mesh=mesh,
        out_type=jax.ShapeDtypeStruct((nb, D), jnp.float32),
        scratch_types=[
            pltpu.VMEM((bpw,), jnp.int32),
            [pltpu.VMEM((C, D), jnp.float32) for _ in range(N)],
            [pltpu.SemaphoreType.DMA for _ in range(N)],
            [pltpu.SemaphoreType.DMA for _ in range(N)],
        ],
        compiler_params=pltpu.CompilerParams(use_tc_tiling_on_sc=False),
    )
    def k(idx_hbm, table_hbm, out_hbm, idx_all, rows, sg, sw):
        wid = lax.axis_index("s") * nc + lax.axis_index("c")
        base = wid * bpw
        pltpu.sync_copy(idx_hbm.at[pl.ds(base, bpw)], idx_all)

        def gather(j, buf, sem):
            pltpu.async_copy(table_hbm.at[idx_all.at[pl.ds(j * C, C)]], buf, sem)

        def wait_gather(buf, sem):
            pltpu.make_async_copy(table_hbm.at[idx_all.at[pl.ds(0, C)]], buf, sem).wait()

        def write(j, buf, sem):
            pltpu.async_copy(buf, out_hbm.at[pl.ds(base + j * C, C)], sem)

        def wait_write(j, buf, sem):
            pltpu.make_async_copy(buf, out_hbm.at[pl.ds(base + j * C, C)], sem).wait()

        for s in range(N):
            gather(s, rows[s], sg[s])

        def body(t, carry):
            for s in range(N):
                j = N * t + s
                wait_gather(rows[s], sg[s])
                write(j, rows[s], sw[s])

                @pl.when(t < nrounds - 1)
                def _():
                    wait_write(j, rows[s], sw[s])
                    gather(j + N, rows[s], sg[s])

            return carry

        lax.fori_loop(0, nrounds, body, 0, unroll=False)
        for s in range(N):
            wait_write(N * (nrounds - 1) + s, rows[s], sw[s])

    return k(flat_idx, table)


def kernel(input_tokens, table):
    B, H = input_tokens.shape
    D = table.shape[1]
    info = plsc.get_sparse_core_info()
    nc, ns = info.num_cores, info.num_subcores
    nb = B * H
    nw = nc * ns
    bpw = nb // nw
    nrounds = bpw // (_NSLOTS * _CHUNK)
    flat = input_tokens.reshape(nb).astype(jnp.int32)
    out = _sc_gather(flat, table, nb=nb, nc=nc, ns=ns, bpw=bpw, nrounds=nrounds)
    return out.reshape(B, H, D)
